# baseline (speedup 1.0000x reference)
import functools

import jax
import jax.numpy as jnp
from jax import lax
from jax.experimental import pallas as pl
from jax.experimental.pallas import tpu as pltpu
from jax.experimental.pallas import tpu_sc as plsc

BB = 256


def _make_router(T, D, E, NB, interpret=False):
    def body(x_ref, gw_ref, pe_ref, po_ref, w0w_ref, w1w_ref, be_ref,
             nbu_ref, ro_ref, pf_ref, fb_ref, pf2_ref):
        xb = x_ref[...]
        gw = gw_ref[...]
        logits = lax.dot_general(xb, gw, (((1,), (1,)), ((), ())),
                                 preferred_element_type=jnp.float32)
        m = jnp.max(logits, axis=1, keepdims=True)
        ex = jnp.exp(logits - m)
        p = ex / jnp.sum(ex, axis=1, keepdims=True)
        io = lax.broadcasted_iota(jnp.int32, (T, E), 1)
        m1 = jnp.max(p, axis=1, keepdims=True)
        e1 = jnp.min(jnp.where(p == m1, io, E), axis=1, keepdims=True)
        pm = jnp.where(io == e1, jnp.float32(-1.0), p)
        m2 = jnp.max(pm, axis=1, keepdims=True)
        e2 = jnp.min(jnp.where(pm == m2, io, E), axis=1, keepdims=True)
        s = m1 + m2
        w0w_ref[...] = jnp.broadcast_to(m1 / s, (T, 128))
        w1w_ref[...] = jnp.broadcast_to(m2 / s, (T, 128))

        oh1 = (io == e1).astype(jnp.float32)
        oh2 = (io == e2).astype(jnp.float32)
        C = min(T, 512)
        irc = lax.broadcasted_iota(jnp.int32, (C, C), 0)
        icc = lax.broadcasted_iota(jnp.int32, (C, C), 1)
        ltc = (irc >= icc).astype(jnp.float32)
        oh12 = oh1 + oh2
        pieces = []
        off = jnp.zeros((1, E), jnp.float32)
        for c in range(T // C):
            seg = lax.slice(oh12, (c * C, 0), ((c + 1) * C, E))
            part = lax.dot_general(ltc, seg, (((1,), (0,)), ((), ())),
                                   preferred_element_type=jnp.float32)
            pieces.append(part + off)
            off = off + lax.slice(part, (C - 1, 0), (C, E))
        cum12 = jnp.concatenate(pieces, axis=0)
        excl = cum12 - oh1 - oh2
        rank1 = jnp.sum(oh1 * excl, axis=1)
        rank2 = jnp.sum(oh2 * (excl + oh1), axis=1)
        counts = lax.slice(cum12, (T - 1, 0), (T, E))
        blocks = jnp.ceil(counts / BB)
        ier = lax.broadcasted_iota(jnp.int32, (E, E), 0)
        iec = lax.broadcasted_iota(jnp.int32, (E, E), 1)
        ut8 = (ier <= iec).astype(jnp.float32)
        bcum = lax.dot_general(blocks, ut8, (((1,), (0,)), ((), ())),
                               preferred_element_type=jnp.float32)
        poff = (bcum - blocks) * BB
        pe_ref[...] = (jnp.sum(oh1 * poff, axis=1) + rank1).astype(jnp.int32)
        po_ref[...] = (jnp.sum(oh2 * poff, axis=1) + rank2).astype(jnp.int32)
        ib = lax.broadcasted_iota(jnp.int32, (NB, E), 0)
        be = jnp.sum((ib >= bcum.astype(jnp.int32)).astype(jnp.int32),
                     axis=1, keepdims=True)
        be = jnp.minimum(be, E - 1)
        be_ref[...] = be
        nbu = lax.slice(bcum, (0, E - 1), (1, E)).astype(jnp.int32)
        nbu_ref[...] = nbu

        present = (counts > 0).astype(jnp.float32)
        rinc = lax.dot_general(present, ut8, (((1,), (0,)), ((), ())),
                               preferred_element_type=jnp.float32)
        runidx = (rinc - present).astype(jnp.int32)
        ibe = lax.broadcasted_iota(jnp.int32, (NB, E), 1)
        onehot = (ibe == be).astype(jnp.float32)
        ro = jnp.sum(onehot * runidx.astype(jnp.float32), axis=1,
                     keepdims=True).astype(jnp.int32)
        ro_ref[...] = ro
        bstart = bcum - blocks
        ivec = lax.broadcasted_iota(jnp.int32, (NB, 1), 0)
        fb = ((ivec == jnp.sum(onehot * bstart, axis=1,
                               keepdims=True).astype(jnp.int32))
              & (ivec < nbu)).astype(jnp.int32)
        fb_ref[...] = fb
        pres_row = jnp.broadcast_to(present > 0, (NB, E))
        rid_row = jnp.broadcast_to(runidx, (NB, E))
        iecol = lax.broadcasted_iota(jnp.int32, (NB, E), 1)

        def run_expert(target):
            match = (rid_row == target) & pres_row
            val = jnp.sum(jnp.where(match, iecol, 0).astype(jnp.float32),
                          axis=1, keepdims=True).astype(jnp.int32)
            has = jnp.sum(match.astype(jnp.int32), axis=1, keepdims=True) > 0
            return jnp.where((fb == 1) & has, val, -1)

        pf_ref[...] = run_expert(ro + 1)
        pf2_ref[...] = run_expert(ro + 2)

    return pl.pallas_call(
        body,
        out_shape=(jax.ShapeDtypeStruct((T,), jnp.int32),
                   jax.ShapeDtypeStruct((T,), jnp.int32),
                   jax.ShapeDtypeStruct((T, 128), jnp.float32),
                   jax.ShapeDtypeStruct((T, 128), jnp.float32),
                   jax.ShapeDtypeStruct((NB, 1), jnp.int32),
                   jax.ShapeDtypeStruct((1, 1), jnp.int32),
                   jax.ShapeDtypeStruct((NB, 1), jnp.int32),
                   jax.ShapeDtypeStruct((NB, 1), jnp.int32),
                   jax.ShapeDtypeStruct((NB, 1), jnp.int32),
                   jax.ShapeDtypeStruct((NB, 1), jnp.int32)),
        interpret=interpret,
    )


def _make_ffn(NB, D, F, interpret=False):
    def body(be_ref, nbu_ref, ro_ref, pf_ref, fb_ref, pf2_ref,
             xs_ref, w1_ref, w2_ref, w3_ref, rw_ref, out_ref,
             w1a, w2a, w3a, w1b, w2b, w3b, sa13, sa2, sb13, sb2):
        i = pl.program_id(0)
        valid = i < nbu_ref[0, 0]
        e = be_ref[i, 0]
        p = lax.rem(ro_ref[i, 0], 2)
        fb = fb_ref[i, 0] == 1
        nx = pf_ref[i, 0]
        bufs = ((w1a, w2a, w3a, sa13, sa2), (w1b, w2b, w3b, sb13, sb2))

        def issue(eidx, bset):
            d1, d2, d3, s13, s2 = bset
            pltpu.make_async_copy(w1_ref.at[eidx], d1, s13).start()
            pltpu.make_async_copy(w3_ref.at[eidx], d3, s13).start()
            pltpu.make_async_copy(w2_ref.at[eidx], d2, s2).start()

        @pl.when(i == 0)
        def _():
            issue(e, bufs[0])

        for q in range(2):
            @pl.when(valid & fb & (nx >= 0) & (p == q))
            def _(q=q):
                issue(nx, bufs[1 - q])

        def compute(bset):
            d1, d2, d3, s13, s2 = bset

            @pl.when(fb)
            def _():
                pltpu.make_async_copy(w1_ref.at[e], d1, s13).wait()
                pltpu.make_async_copy(w3_ref.at[e], d3, s13).wait()

            xb = xs_ref[...]
            h1 = lax.dot_general(xb, d1[...], (((1,), (1,)), ((), ())),
                                 preferred_element_type=jnp.float32)
            h3 = lax.dot_general(xb, d3[...], (((1,), (1,)), ((), ())),
                                 preferred_element_type=jnp.float32)
            h = jnp.maximum(h1, 0.0) * h3

            @pl.when(fb)
            def _():
                pltpu.make_async_copy(w2_ref.at[e], d2, s2).wait()

            ob = lax.dot_general(h, d2[...], (((1,), (1,)), ((), ())),
                                 preferred_element_type=jnp.float32)
            out_ref[...] = ob * rw_ref[:, :1]

        for q in range(2):
            @pl.when(valid & (p == q))
            def _(q=q):
                compute(bufs[q])

    grid_spec = pltpu.PrefetchScalarGridSpec(
        num_scalar_prefetch=6,
        grid=(NB,),
        in_specs=[
            pl.BlockSpec((BB, D), lambda i, *refs: (i, 0)),
            pl.BlockSpec(memory_space=pltpu.MemorySpace.HBM),
            pl.BlockSpec(memory_space=pltpu.MemorySpace.HBM),
            pl.BlockSpec(memory_space=pltpu.MemorySpace.HBM),
            pl.BlockSpec((BB, 128), lambda i, *refs: (i, 0)),
        ],
        out_specs=pl.BlockSpec((BB, D), lambda i, *refs: (i, 0)),
        scratch_shapes=[
            pltpu.VMEM((F, D), jnp.float32),
            pltpu.VMEM((D, F), jnp.float32),
            pltpu.VMEM((F, D), jnp.float32),
            pltpu.VMEM((F, D), jnp.float32),
            pltpu.VMEM((D, F), jnp.float32),
            pltpu.VMEM((F, D), jnp.float32),
            pltpu.SemaphoreType.DMA,
            pltpu.SemaphoreType.DMA,
            pltpu.SemaphoreType.DMA,
            pltpu.SemaphoreType.DMA,
        ],
    )
    return pl.pallas_call(
        body,
        grid_spec=grid_spec,
        out_shape=jax.ShapeDtypeStruct((NB * BB, D), jnp.float32),
        compiler_params=pltpu.CompilerParams(
            vmem_limit_bytes=110 * 1024 * 1024),
        interpret=interpret,
    )


def _make_sc_dispatch(T, D, PT):
    info = plsc.get_sparse_core_info()
    NC, NS = info.num_cores, info.num_subcores
    NW = NC * NS
    TW = T // NW
    CT = 32
    NCH = TW // CT
    mesh = plsc.VectorSubcoreMesh(core_axis_name="c", subcore_axis_name="s")

    @functools.partial(
        pl.kernel, mesh=mesh,
        out_type=(jax.ShapeDtypeStruct((PT, D), jnp.float32),
                  jax.ShapeDtypeStruct((PT, 128), jnp.float32)),
        scratch_types=[
            pltpu.VMEM((NCH, CT), jnp.int32),
            pltpu.VMEM((NCH, CT), jnp.int32),
            pltpu.VMEM((CT, D), jnp.float32),
            pltpu.VMEM((CT, D), jnp.float32),
            pltpu.VMEM((TW, 128), jnp.float32),
            pltpu.VMEM((TW, 128), jnp.float32),
            pltpu.SemaphoreType.DMA,
            pltpu.SemaphoreType.DMA,
            pltpu.SemaphoreType.DMA,
        ],
    )
    def k(x_hbm, pe_hbm, po_hbm, w0_hbm, w1_hbm, xs_hbm, rw_hbm,
          pe_v, po_v, buf0, buf1, w0_v, w1_v, seml0, seml1, sems):
        wid = lax.axis_index("s") * NC + lax.axis_index("c")
        base = wid * TW
        for c in range(NCH):
            pltpu.sync_copy(pe_hbm.at[pl.ds(base + c * CT, CT)], pe_v.at[c])
            pltpu.sync_copy(po_hbm.at[pl.ds(base + c * CT, CT)], po_v.at[c])
        pltpu.sync_copy(w0_hbm.at[pl.ds(base, TW)], w0_v)
        pltpu.sync_copy(w1_hbm.at[pl.ds(base, TW)], w1_v)
        bufs = (buf0, buf1)
        semls = (seml0, seml1)
        loads = [None] * NCH
        for c in range(min(2, NCH)):
            loads[c] = pltpu.async_copy(
                x_hbm.at[pl.ds(base + c * CT, CT)], bufs[c % 2],
                semls[c % 2])
        scats = []
        for c in range(NCH):
            loads[c].wait()
            b = bufs[c % 2]
            scats.append(pltpu.async_copy(b, xs_hbm.at[pe_v.at[c]], sems))
            scats.append(pltpu.async_copy(b, xs_hbm.at[po_v.at[c]], sems))
            scats.append(pltpu.async_copy(w0_v.at[pl.ds(c * CT, CT)],
                                          rw_hbm.at[pe_v.at[c]], sems))
            scats.append(pltpu.async_copy(w1_v.at[pl.ds(c * CT, CT)],
                                          rw_hbm.at[po_v.at[c]], sems))
            if c + 2 < NCH:
                for s in scats:
                    s.wait()
                scats = []
                loads[c + 2] = pltpu.async_copy(
                    x_hbm.at[pl.ds(base + (c + 2) * CT, CT)],
                    bufs[c % 2], semls[c % 2])
        for s in scats:
            s.wait()

    return k


def _make_sc_combine(T, D, PT):
    info = plsc.get_sparse_core_info()
    NC, NS = info.num_cores, info.num_subcores
    NW = NC * NS
    TW = T // NW
    CT = 16
    NCH = TW // CT
    mesh = plsc.VectorSubcoreMesh(core_axis_name="c", subcore_axis_name="s")

    @functools.partial(
        pl.kernel, mesh=mesh,
        out_type=jax.ShapeDtypeStruct((T, D), jnp.float32),
        scratch_types=[
            pltpu.VMEM((TW,), jnp.int32),
            pltpu.VMEM((TW,), jnp.int32),
            pltpu.VMEM((CT, D), jnp.float32),
            pltpu.VMEM((CT, D), jnp.float32),
            pltpu.VMEM((CT, D), jnp.float32),
            pltpu.VMEM((CT, D), jnp.float32),
            pltpu.VMEM((CT, D), jnp.float32),
            pltpu.SemaphoreType.DMA,
            pltpu.SemaphoreType.DMA,
        ],
    )
    def k(ys_hbm, pe_hbm, po_hbm, out_hbm,
          pe_v, po_v, ga0, gb0, ga1, gb1, o_v, sem0, sem1):
        wid = lax.axis_index("s") * NC + lax.axis_index("c")
        base = wid * TW
        pltpu.sync_copy(pe_hbm.at[pl.ds(base, TW)], pe_v)
        pltpu.sync_copy(po_hbm.at[pl.ds(base, TW)], po_v)
        gas = (ga0, ga1)
        gbs = (gb0, gb1)
        sems = (sem0, sem1)
        cps = [None] * NCH
        for c in range(min(2, NCH)):
            cps[c] = (
                pltpu.async_copy(ys_hbm.at[pe_v.at[pl.ds(c * CT, CT)]],
                                 gas[c % 2], sems[c % 2]),
                pltpu.async_copy(ys_hbm.at[po_v.at[pl.ds(c * CT, CT)]],
                                 gbs[c % 2], sems[c % 2]))
        for c in range(NCH):
            cps[c][0].wait()
            cps[c][1].wait()
            ga, gb = gas[c % 2], gbs[c % 2]

            def row_body(j, _, ga=ga, gb=gb):
                for c2 in range(D // 16):
                    sl = pl.ds(c2 * 16, 16)
                    o_v[j, sl] = ga[j, sl] + gb[j, sl]
                return 0

            lax.fori_loop(0, CT, row_body, 0)
            pltpu.sync_copy(o_v, out_hbm.at[pl.ds(base + c * CT, CT)])
            if c + 2 < NCH:
                cps[c + 2] = (
                    pltpu.async_copy(
                        ys_hbm.at[pe_v.at[pl.ds((c + 2) * CT, CT)]],
                        gas[c % 2], sems[c % 2]),
                    pltpu.async_copy(
                        ys_hbm.at[po_v.at[pl.ds((c + 2) * CT, CT)]],
                        gbs[c % 2], sems[c % 2]))

    return k


def kernel(x, gate_w, w1, w2, w3):
    Bb, S, D = x.shape
    T = Bb * S
    E, F, _ = w1.shape
    K = 2
    NP = T * K
    NB = NP // BB + E
    PT = NB * BB

    xf = x.reshape(T, D)

    pe, po, w0w, w1w, be, nbu, ro, pf, fb, pf2 = _make_router(T, D, E, NB)(
        xf, gate_w)

    xs, rw = _make_sc_dispatch(T, D, PT)(xf, pe, po, w0w, w1w)

    ys = _make_ffn(NB, D, F)(be, nbu, ro, pf, fb, pf2, xs, w1, w2, w3, rw)

    out = _make_sc_combine(T, D, PT)(ys, pe, po)
    return out.reshape(Bb, S, D)


# speedup vs baseline: 1.0135x; 1.0135x over previous
import functools

import jax
import jax.numpy as jnp
from jax import lax
from jax.experimental import pallas as pl
from jax.experimental.pallas import tpu as pltpu
from jax.experimental.pallas import tpu_sc as plsc

BB = 256


def _make_router(T, D, E, NB, interpret=False):
    def body(x_ref, gw_ref, pe_ref, po_ref, w0w_ref, w1w_ref, be_ref,
             nbu_ref, ro_ref, pf_ref, fb_ref, pf2_ref):
        xb = x_ref[...]
        gw = gw_ref[...]
        logits = lax.dot_general(xb, gw, (((1,), (1,)), ((), ())),
                                 preferred_element_type=jnp.float32)
        m = jnp.max(logits, axis=1, keepdims=True)
        ex = jnp.exp(logits - m)
        p = ex / jnp.sum(ex, axis=1, keepdims=True)
        io = lax.broadcasted_iota(jnp.int32, (T, E), 1)
        m1 = jnp.max(p, axis=1, keepdims=True)
        e1 = jnp.min(jnp.where(p == m1, io, E), axis=1, keepdims=True)
        pm = jnp.where(io == e1, jnp.float32(-1.0), p)
        m2 = jnp.max(pm, axis=1, keepdims=True)
        e2 = jnp.min(jnp.where(pm == m2, io, E), axis=1, keepdims=True)
        s = m1 + m2
        w0w_ref[...] = jnp.broadcast_to(m1 / s, (T, 128))
        w1w_ref[...] = jnp.broadcast_to(m2 / s, (T, 128))

        oh1 = (io == e1).astype(jnp.float32)
        oh2 = (io == e2).astype(jnp.float32)
        C = min(T, 512)
        irc = lax.broadcasted_iota(jnp.int32, (C, C), 0)
        icc = lax.broadcasted_iota(jnp.int32, (C, C), 1)
        ltc = (irc >= icc).astype(jnp.float32)
        oh12 = oh1 + oh2
        pieces = []
        off = jnp.zeros((1, E), jnp.float32)
        for c in range(T // C):
            seg = lax.slice(oh12, (c * C, 0), ((c + 1) * C, E))
            part = lax.dot_general(ltc, seg, (((1,), (0,)), ((), ())),
                                   preferred_element_type=jnp.float32)
            pieces.append(part + off)
            off = off + lax.slice(part, (C - 1, 0), (C, E))
        cum12 = jnp.concatenate(pieces, axis=0)
        excl = cum12 - oh1 - oh2
        rank1 = jnp.sum(oh1 * excl, axis=1)
        rank2 = jnp.sum(oh2 * (excl + oh1), axis=1)
        counts = lax.slice(cum12, (T - 1, 0), (T, E))
        blocks = jnp.ceil(counts / BB)
        ier = lax.broadcasted_iota(jnp.int32, (E, E), 0)
        iec = lax.broadcasted_iota(jnp.int32, (E, E), 1)
        ut8 = (ier <= iec).astype(jnp.float32)
        bcum = lax.dot_general(blocks, ut8, (((1,), (0,)), ((), ())),
                               preferred_element_type=jnp.float32)
        poff = (bcum - blocks) * BB
        pe_ref[...] = (jnp.sum(oh1 * poff, axis=1) + rank1).astype(jnp.int32)
        po_ref[...] = (jnp.sum(oh2 * poff, axis=1) + rank2).astype(jnp.int32)
        ib = lax.broadcasted_iota(jnp.int32, (NB, E), 0)
        be = jnp.sum((ib >= bcum.astype(jnp.int32)).astype(jnp.int32),
                     axis=1, keepdims=True)
        be = jnp.minimum(be, E - 1)
        be_ref[...] = be
        nbu = lax.slice(bcum, (0, E - 1), (1, E)).astype(jnp.int32)
        nbu_ref[...] = nbu

        present = (counts > 0).astype(jnp.float32)
        rinc = lax.dot_general(present, ut8, (((1,), (0,)), ((), ())),
                               preferred_element_type=jnp.float32)
        runidx = (rinc - present).astype(jnp.int32)
        ibe = lax.broadcasted_iota(jnp.int32, (NB, E), 1)
        onehot = (ibe == be).astype(jnp.float32)
        ro = jnp.sum(onehot * runidx.astype(jnp.float32), axis=1,
                     keepdims=True).astype(jnp.int32)
        ro_ref[...] = ro
        bstart = bcum - blocks
        ivec = lax.broadcasted_iota(jnp.int32, (NB, 1), 0)
        fb = ((ivec == jnp.sum(onehot * bstart, axis=1,
                               keepdims=True).astype(jnp.int32))
              & (ivec < nbu)).astype(jnp.int32)
        fb_ref[...] = fb
        pres_row = jnp.broadcast_to(present > 0, (NB, E))
        rid_row = jnp.broadcast_to(runidx, (NB, E))
        iecol = lax.broadcasted_iota(jnp.int32, (NB, E), 1)

        def run_expert(target):
            match = (rid_row == target) & pres_row
            val = jnp.sum(jnp.where(match, iecol, 0).astype(jnp.float32),
                          axis=1, keepdims=True).astype(jnp.int32)
            has = jnp.sum(match.astype(jnp.int32), axis=1, keepdims=True) > 0
            return jnp.where((fb == 1) & has, val, -1)

        pf_ref[...] = run_expert(ro + 1)
        pf2_ref[...] = run_expert(ro + 2)

    return pl.pallas_call(
        body,
        out_shape=(jax.ShapeDtypeStruct((T,), jnp.int32),
                   jax.ShapeDtypeStruct((T,), jnp.int32),
                   jax.ShapeDtypeStruct((T, 128), jnp.float32),
                   jax.ShapeDtypeStruct((T, 128), jnp.float32),
                   jax.ShapeDtypeStruct((NB, 1), jnp.int32),
                   jax.ShapeDtypeStruct((1, 1), jnp.int32),
                   jax.ShapeDtypeStruct((NB, 1), jnp.int32),
                   jax.ShapeDtypeStruct((NB, 1), jnp.int32),
                   jax.ShapeDtypeStruct((NB, 1), jnp.int32),
                   jax.ShapeDtypeStruct((NB, 1), jnp.int32)),
        interpret=interpret,
    )


def _make_ffn(NB, D, F, interpret=False):
    def body(be_ref, nbu_ref, ro_ref, pf_ref, fb_ref, pf2_ref,
             xs_ref, w1_ref, w2_ref, w3_ref, rw_ref, out_ref,
             w1a, w2a, w3a, w1b, w2b, w3b, sa13, sa2, sb13, sb2):
        i = pl.program_id(0)
        valid = i < nbu_ref[0, 0]
        e = be_ref[i, 0]
        p = lax.rem(ro_ref[i, 0], 2)
        fb = fb_ref[i, 0] == 1
        nx = pf_ref[i, 0]
        bufs = ((w1a, w2a, w3a, sa13, sa2), (w1b, w2b, w3b, sb13, sb2))

        def issue(eidx, bset):
            d1, d2, d3, s13, s2 = bset
            pltpu.make_async_copy(w1_ref.at[eidx], d1, s13).start()
            pltpu.make_async_copy(w3_ref.at[eidx], d3, s13).start()
            pltpu.make_async_copy(w2_ref.at[eidx], d2, s2).start()

        def drain(eidx, bset):
            d1, d2, d3, s13, s2 = bset
            pltpu.make_async_copy(w1_ref.at[eidx], d1, s13).wait()
            pltpu.make_async_copy(w3_ref.at[eidx], d3, s13).wait()
            pltpu.make_async_copy(w2_ref.at[eidx], d2, s2).wait()

        @pl.when(i == 0)
        def _():
            issue(e, bufs[0])

        for q in range(2):
            @pl.when(valid & fb & (p == q))
            def _(q=q):
                drain(e, bufs[q])

            @pl.when(valid & fb & (nx >= 0) & (p == q))
            def _(q=q):
                issue(nx, bufs[1 - q])

        def compute(bset):
            d1, d2, d3, _, _ = bset
            xb = xs_ref[...]
            h1 = lax.dot_general(xb, d1[...], (((1,), (1,)), ((), ())),
                                 preferred_element_type=jnp.float32)
            h3 = lax.dot_general(xb, d3[...], (((1,), (1,)), ((), ())),
                                 preferred_element_type=jnp.float32)
            h = jnp.maximum(h1, 0.0) * h3
            ob = lax.dot_general(h, d2[...], (((1,), (1,)), ((), ())),
                                 preferred_element_type=jnp.float32)
            out_ref[...] = ob * rw_ref[:, :1]

        for q in range(2):
            @pl.when(valid & (p == q))
            def _(q=q):
                compute(bufs[q])

    grid_spec = pltpu.PrefetchScalarGridSpec(
        num_scalar_prefetch=6,
        grid=(NB,),
        in_specs=[
            pl.BlockSpec((BB, D), lambda i, *refs: (i, 0)),
            pl.BlockSpec(memory_space=pltpu.MemorySpace.HBM),
            pl.BlockSpec(memory_space=pltpu.MemorySpace.HBM),
            pl.BlockSpec(memory_space=pltpu.MemorySpace.HBM),
            pl.BlockSpec((BB, 128), lambda i, *refs: (i, 0)),
        ],
        out_specs=pl.BlockSpec((BB, D), lambda i, *refs: (i, 0)),
        scratch_shapes=[
            pltpu.VMEM((F, D), jnp.float32),
            pltpu.VMEM((D, F), jnp.float32),
            pltpu.VMEM((F, D), jnp.float32),
            pltpu.VMEM((F, D), jnp.float32),
            pltpu.VMEM((D, F), jnp.float32),
            pltpu.VMEM((F, D), jnp.float32),
            pltpu.SemaphoreType.DMA,
            pltpu.SemaphoreType.DMA,
            pltpu.SemaphoreType.DMA,
            pltpu.SemaphoreType.DMA,
        ],
    )
    return pl.pallas_call(
        body,
        grid_spec=grid_spec,
        out_shape=jax.ShapeDtypeStruct((NB * BB, D), jnp.float32),
        compiler_params=pltpu.CompilerParams(
            vmem_limit_bytes=110 * 1024 * 1024),
        interpret=interpret,
    )


def _make_sc_dispatch(T, D, PT):
    info = plsc.get_sparse_core_info()
    NC, NS = info.num_cores, info.num_subcores
    NW = NC * NS
    TW = T // NW
    CT = 32
    NCH = TW // CT
    mesh = plsc.VectorSubcoreMesh(core_axis_name="c", subcore_axis_name="s")

    @functools.partial(
        pl.kernel, mesh=mesh,
        out_type=(jax.ShapeDtypeStruct((PT, D), jnp.float32),
                  jax.ShapeDtypeStruct((PT, 128), jnp.float32)),
        scratch_types=[
            pltpu.VMEM((NCH, CT), jnp.int32),
            pltpu.VMEM((NCH, CT), jnp.int32),
            pltpu.VMEM((CT, D), jnp.float32),
            pltpu.VMEM((CT, D), jnp.float32),
            pltpu.VMEM((TW, 128), jnp.float32),
            pltpu.VMEM((TW, 128), jnp.float32),
            pltpu.SemaphoreType.DMA,
            pltpu.SemaphoreType.DMA,
            pltpu.SemaphoreType.DMA,
        ],
    )
    def k(x_hbm, pe_hbm, po_hbm, w0_hbm, w1_hbm, xs_hbm, rw_hbm,
          pe_v, po_v, buf0, buf1, w0_v, w1_v, seml0, seml1, sems):
        wid = lax.axis_index("s") * NC + lax.axis_index("c")
        base = wid * TW
        for c in range(NCH):
            pltpu.sync_copy(pe_hbm.at[pl.ds(base + c * CT, CT)], pe_v.at[c])
            pltpu.sync_copy(po_hbm.at[pl.ds(base + c * CT, CT)], po_v.at[c])
        pltpu.sync_copy(w0_hbm.at[pl.ds(base, TW)], w0_v)
        pltpu.sync_copy(w1_hbm.at[pl.ds(base, TW)], w1_v)
        bufs = (buf0, buf1)
        semls = (seml0, seml1)
        loads = [None] * NCH
        for c in range(min(2, NCH)):
            loads[c] = pltpu.async_copy(
                x_hbm.at[pl.ds(base + c * CT, CT)], bufs[c % 2],
                semls[c % 2])
        scats = []
        for c in range(NCH):
            loads[c].wait()
            b = bufs[c % 2]
            scats.append(pltpu.async_copy(b, xs_hbm.at[pe_v.at[c]], sems))
            scats.append(pltpu.async_copy(b, xs_hbm.at[po_v.at[c]], sems))
            scats.append(pltpu.async_copy(w0_v.at[pl.ds(c * CT, CT)],
                                          rw_hbm.at[pe_v.at[c]], sems))
            scats.append(pltpu.async_copy(w1_v.at[pl.ds(c * CT, CT)],
                                          rw_hbm.at[po_v.at[c]], sems))
            if c + 2 < NCH:
                for s in scats:
                    s.wait()
                scats = []
                loads[c + 2] = pltpu.async_copy(
                    x_hbm.at[pl.ds(base + (c + 2) * CT, CT)],
                    bufs[c % 2], semls[c % 2])
        for s in scats:
            s.wait()

    return k


def _make_sc_combine(T, D, PT):
    info = plsc.get_sparse_core_info()
    NC, NS = info.num_cores, info.num_subcores
    NW = NC * NS
    TW = T // NW
    CT = 16
    NCH = TW // CT
    mesh = plsc.VectorSubcoreMesh(core_axis_name="c", subcore_axis_name="s")

    @functools.partial(
        pl.kernel, mesh=mesh,
        out_type=jax.ShapeDtypeStruct((T, D), jnp.float32),
        scratch_types=[
            pltpu.VMEM((TW,), jnp.int32),
            pltpu.VMEM((TW,), jnp.int32),
            pltpu.VMEM((CT, D), jnp.float32),
            pltpu.VMEM((CT, D), jnp.float32),
            pltpu.VMEM((CT, D), jnp.float32),
            pltpu.VMEM((CT, D), jnp.float32),
            pltpu.VMEM((CT, D), jnp.float32),
            pltpu.SemaphoreType.DMA,
            pltpu.SemaphoreType.DMA,
        ],
    )
    def k(ys_hbm, pe_hbm, po_hbm, out_hbm,
          pe_v, po_v, ga0, gb0, ga1, gb1, o_v, sem0, sem1):
        wid = lax.axis_index("s") * NC + lax.axis_index("c")
        base = wid * TW
        pltpu.sync_copy(pe_hbm.at[pl.ds(base, TW)], pe_v)
        pltpu.sync_copy(po_hbm.at[pl.ds(base, TW)], po_v)
        gas = (ga0, ga1)
        gbs = (gb0, gb1)
        sems = (sem0, sem1)
        cps = [None] * NCH
        for c in range(min(2, NCH)):
            cps[c] = (
                pltpu.async_copy(ys_hbm.at[pe_v.at[pl.ds(c * CT, CT)]],
                                 gas[c % 2], sems[c % 2]),
                pltpu.async_copy(ys_hbm.at[po_v.at[pl.ds(c * CT, CT)]],
                                 gbs[c % 2], sems[c % 2]))
        for c in range(NCH):
            cps[c][0].wait()
            cps[c][1].wait()
            ga, gb = gas[c % 2], gbs[c % 2]

            def row_body(j, _, ga=ga, gb=gb):
                for c2 in range(D // 16):
                    sl = pl.ds(c2 * 16, 16)
                    o_v[j, sl] = ga[j, sl] + gb[j, sl]
                return 0

            lax.fori_loop(0, CT, row_body, 0)
            pltpu.sync_copy(o_v, out_hbm.at[pl.ds(base + c * CT, CT)])
            if c + 2 < NCH:
                cps[c + 2] = (
                    pltpu.async_copy(
                        ys_hbm.at[pe_v.at[pl.ds((c + 2) * CT, CT)]],
                        gas[c % 2], sems[c % 2]),
                    pltpu.async_copy(
                        ys_hbm.at[po_v.at[pl.ds((c + 2) * CT, CT)]],
                        gbs[c % 2], sems[c % 2]))

    return k


def kernel(x, gate_w, w1, w2, w3):
    Bb, S, D = x.shape
    T = Bb * S
    E, F, _ = w1.shape
    K = 2
    NP = T * K
    NB = NP // BB + E
    PT = NB * BB

    xf = x.reshape(T, D)

    pe, po, w0w, w1w, be, nbu, ro, pf, fb, pf2 = _make_router(T, D, E, NB)(
        xf, gate_w)

    xs, rw = _make_sc_dispatch(T, D, PT)(xf, pe, po, w0w, w1w)

    ys = _make_ffn(NB, D, F)(be, nbu, ro, pf, fb, pf2, xs, w1, w2, w3, rw)

    out = _make_sc_combine(T, D, PT)(ys, pe, po)
    return out.reshape(Bb, S, D)
